# baseline (speedup 1.0000x reference)
import jax
import jax.numpy as jnp
from jax import lax
from jax.experimental import pallas as pl

VOCAB = 1000
ROW_BLK = 2048


def _onehot_block(x_ref, o_ref):
    idx = x_ref[...]
    iota = lax.broadcasted_iota(jnp.int32, (ROW_BLK, VOCAB), 1)
    o_ref[...] = (iota == idx[:, None]).astype(jnp.float32)


def kernel(x):
    b, f = x.shape
    n = b * f
    flat = x.reshape(n)
    grid = n // ROW_BLK
    out = pl.pallas_call(
        _onehot_block,
        grid=(grid,),
        in_specs=[pl.BlockSpec((ROW_BLK,), lambda i: (i,))],
        out_specs=pl.BlockSpec((ROW_BLK, VOCAB), lambda i: (i, 0)),
        out_shape=jax.ShapeDtypeStruct((n, VOCAB), jnp.float32),
    )(flat)
    return out.reshape(b, f, VOCAB)


# speedup vs baseline: 1.3844x; 1.3844x over previous
import jax
import jax.numpy as jnp
from jax import lax
from jax.experimental import pallas as pl

VOCAB = 1000
B_BLK = 128


def _onehot_block(x_ref, o_ref):
    idx = x_ref[...]
    iota = lax.broadcasted_iota(jnp.int32, (B_BLK, idx.shape[1], VOCAB), 2)
    o_ref[...] = jnp.where(iota == idx[:, :, None], 1.0, 0.0).astype(jnp.float32)


def kernel(x):
    b, f = x.shape
    grid = b // B_BLK
    return pl.pallas_call(
        _onehot_block,
        grid=(grid,),
        in_specs=[pl.BlockSpec((B_BLK, f), lambda i: (i, 0))],
        out_specs=pl.BlockSpec((B_BLK, f, VOCAB), lambda i: (i, 0, 0)),
        out_shape=jax.ShapeDtypeStruct((b, f, VOCAB), jnp.float32),
    )(x)


# speedup vs baseline: 1.3881x; 1.0027x over previous
import jax
import jax.numpy as jnp
from jax import lax
from jax.experimental import pallas as pl
from jax.experimental.pallas import tpu as pltpu

VOCAB = 1000
B_BLK = 32
NBUF = 8


def _onehot_block(x_ref, o_hbm, scratch, sems):
    i = pl.program_id(0)
    g = pl.num_programs(0)
    buf = lax.rem(i, NBUF)
    f = x_ref.shape[1]

    @pl.when(i >= NBUF)
    def _():
        pltpu.make_async_copy(
            scratch.at[buf],
            o_hbm.at[pl.ds((i - NBUF) * B_BLK, B_BLK)],
            sems.at[buf],
        ).wait()

    idx = x_ref[...]
    iota = lax.broadcasted_iota(jnp.int32, (B_BLK, f, VOCAB), 2)
    scratch[buf] = jnp.where(iota == idx[:, :, None], 1.0, 0.0).astype(jnp.float32)

    pltpu.make_async_copy(
        scratch.at[buf],
        o_hbm.at[pl.ds(i * B_BLK, B_BLK)],
        sems.at[buf],
    ).start()

    @pl.when(i == g - 1)
    def _():
        for j in range(NBUF):
            pltpu.make_async_copy(
                scratch.at[j],
                o_hbm.at[pl.ds((g - NBUF + j) * B_BLK, B_BLK)],
                sems.at[j],
            ).wait()


def kernel(x):
    b, f = x.shape
    grid = b // B_BLK
    return pl.pallas_call(
        _onehot_block,
        grid=(grid,),
        in_specs=[pl.BlockSpec((B_BLK, f), lambda i: (i, 0))],
        out_specs=pl.BlockSpec(memory_space=pl.ANY),
        out_shape=jax.ShapeDtypeStruct((b, f, VOCAB), jnp.float32),
        scratch_shapes=[
            pltpu.VMEM((NBUF, B_BLK, f, VOCAB), jnp.float32),
            pltpu.SemaphoreType.DMA((NBUF,)),
        ],
    )(x)
